# baseline (speedup 1.0000x reference)
import functools

import jax
import jax.numpy as jnp
from jax import lax
from jax.experimental import pallas as pl
from jax.experimental.pallas import tpu as pltpu
from jax.experimental.pallas import tpu_sc as plsc

_NUM_EMB = 1024
_EDIM = 64
_BETA = 0.25

_info = plsc.get_sparse_core_info()
_NC, _NS = _info.num_cores, _info.num_subcores
_NW = _NC * _NS


def _phase_matmul(x_ref, e, sc_w, xsq_w):
    x = x_ref[0]
    sc_w[...] = lax.dot_general(
        x, e + e, (((0,), (1,)), ((), ())),
        preferred_element_type=jnp.float32)
    xsq_w[...] = jnp.sum(x * x, axis=0).reshape(1, -1)


def _phase_argmin(e, sc_r, xsq_r, idx_ref, loss_ref, s):
    esq = jnp.sum(e * e, axis=1)
    xsq = xsq_r[0]
    d = (xsq[:, None] + esq[None, :]) - sc_r[...]

    m = jnp.min(d, axis=1)
    col_iota = lax.broadcasted_iota(jnp.int32, d.shape, 1)
    idx = jnp.min(jnp.where(d == m[:, None], col_iota, _NUM_EMB), axis=1)

    idx_ref[0, 0] = idx
    contrib = jnp.where(s > 0, jnp.sum(m), 0.0)
    loss_ref[...] += contrib.reshape(1, 1)


def _argmin_body(x_ref, e_ref, idx_ref, loss_ref, sc_a, sc_b, xsq_a, xsq_b):
    s = pl.program_id(0)
    e = e_ref[...]

    @pl.when(s == 0)
    def _():
        loss_ref[...] = jnp.zeros((1, 1), jnp.float32)

    @pl.when(s % 2 == 0)
    def _():
        _phase_matmul(x_ref, e, sc_a, xsq_a)
        _phase_argmin(e, sc_b, xsq_b, idx_ref, loss_ref, s)

    @pl.when(s % 2 == 1)
    def _():
        _phase_matmul(x_ref, e, sc_b, xsq_b)
        _phase_argmin(e, sc_a, xsq_a, idx_ref, loss_ref, s)


def _argmin_call(x, embeddings):
    B = x.shape[0]
    hw = x.shape[2]
    return pl.pallas_call(
        _argmin_body,
        grid=(B + 1,),
        in_specs=[
            pl.BlockSpec((1, _EDIM, hw), lambda s: (jnp.minimum(s, B - 1), 0, 0)),
            pl.BlockSpec((_NUM_EMB, _EDIM), lambda s: (0, 0)),
        ],
        out_specs=[
            pl.BlockSpec((1, 1, hw), lambda s: (jnp.maximum(s - 1, 0), 0, 0)),
            pl.BlockSpec((1, 1), lambda s: (0, 0)),
        ],
        out_shape=[
            jax.ShapeDtypeStruct((B, 1, hw), jnp.int32),
            jax.ShapeDtypeStruct((1, 1), jnp.float32),
        ],
        scratch_shapes=[
            pltpu.VMEM((hw, _NUM_EMB), jnp.float32),
            pltpu.VMEM((hw, _NUM_EMB), jnp.float32),
            pltpu.VMEM((1, hw), jnp.float32),
            pltpu.VMEM((1, hw), jnp.float32),
        ],
    )(x, embeddings)


def _make_sc_gather(B, hw):
    n_rows = B * hw
    bpw = n_rows // _NW
    mesh = plsc.VectorSubcoreMesh(core_axis_name="c", subcore_axis_name="s")

    @functools.partial(
        pl.kernel,
        mesh=mesh,
        out_type=jax.ShapeDtypeStruct((B, _EDIM, hw), jnp.float32),
        compiler_params=pltpu.CompilerParams(needs_layout_passes=False),
        scratch_types=[
            pltpu.VMEM((_NUM_EMB * _EDIM,), jnp.float32),
            pltpu.VMEM((bpw,), jnp.int32),
            pltpu.VMEM((_EDIM, bpw), jnp.float32),
            pltpu.SemaphoreType.DMA,
            pltpu.SemaphoreType.DMA,
        ],
    )
    def gather_k(table_hbm, idx_hbm, out_hbm, tab_v, idx_v, out_v, sem_t, sem_i):
        wid = lax.axis_index("s") * _NC + lax.axis_index("c")
        base = wid * bpw
        b = base // hw
        off = base % hw
        cp_t = pltpu.async_copy(table_hbm, tab_v, sem_t)
        cp_i = pltpu.async_copy(idx_hbm.at[pl.ds(base, bpw)], idx_v, sem_i)
        cp_t.wait()
        cp_i.wait()

        iota16 = lax.iota(jnp.int32, 16)

        @plsc.parallel_loop(0, bpw // 16, unroll=4)
        def _(jc):
            idx16 = idx_v[pl.ds(jc * 16, 16)]
            rowa = idx16 * _EDIM
            jvec = jc * 16 + iota16
            for c in range(_EDIM):
                cvec = (iota16 + c) & (_EDIM - 1)
                vals = plsc.load_gather(tab_v, [rowa + cvec])
                plsc.store_scatter(out_v, [cvec, jvec], vals)
        pltpu.sync_copy(out_v, out_hbm.at[b, :, pl.ds(off, bpw)])

    return gather_k


def kernel(inputs, embeddings):
    B, C, H, W = inputs.shape
    hw = H * W
    x = inputs.reshape(B, C, hw)

    idx3, loss_sum = _argmin_call(x, embeddings)
    idx_flat = idx3.reshape(B * hw)

    q_cm = _make_sc_gather(B, hw)(
        embeddings.reshape(_NUM_EMB * _EDIM), idx_flat)

    quantized = q_cm.reshape(B, C, H, W)
    loss = (1.0 + _BETA) * loss_sum[0, 0] / inputs.size
    return (quantized, loss)


# speedup vs baseline: 1.0158x; 1.0158x over previous
import functools

import jax
import jax.numpy as jnp
from jax import lax
from jax.experimental import pallas as pl
from jax.experimental.pallas import tpu as pltpu
from jax.experimental.pallas import tpu_sc as plsc

_NUM_EMB = 1024
_EDIM = 64
_BETA = 0.25

_info = plsc.get_sparse_core_info()
_NC, _NS = _info.num_cores, _info.num_subcores
_NW = _NC * _NS


def _argmin_body(x_ref, e_ref, idx_ref, loss_ref):
    b = pl.program_id(0)
    x = x_ref[0]
    e = e_ref[...]

    scores2 = lax.dot_general(
        x, e + e, (((0,), (1,)), ((), ())),
        preferred_element_type=jnp.float32)
    esq = jnp.sum(e * e, axis=1)
    xsq = jnp.sum(x * x, axis=0)
    d = (xsq[:, None] + esq[None, :]) - scores2

    m = jnp.min(d, axis=1)
    col_iota = lax.broadcasted_iota(jnp.int32, d.shape, 1)
    idx = jnp.min(jnp.where(d == m[:, None], col_iota, _NUM_EMB), axis=1)

    idx_ref[0, 0] = idx

    @pl.when(b == 0)
    def _():
        loss_ref[...] = jnp.zeros((1, 1), jnp.float32)
    loss_ref[...] += jnp.sum(m).reshape(1, 1)


def _argmin_call(x, embeddings):
    B = x.shape[0]
    hw = x.shape[2]
    return pl.pallas_call(
        _argmin_body,
        grid=(B,),
        in_specs=[
            pl.BlockSpec((1, _EDIM, hw), lambda b: (b, 0, 0)),
            pl.BlockSpec((_NUM_EMB, _EDIM), lambda b: (0, 0)),
        ],
        out_specs=[
            pl.BlockSpec((1, 1, hw), lambda b: (b, 0, 0)),
            pl.BlockSpec((1, 1), lambda b: (0, 0)),
        ],
        out_shape=[
            jax.ShapeDtypeStruct((B, 1, hw), jnp.int32),
            jax.ShapeDtypeStruct((1, 1), jnp.float32),
        ],
    )(x, embeddings)


def _make_sc_gather(B, hw):
    n_rows = B * hw
    bpw = n_rows // _NW
    mesh = plsc.VectorSubcoreMesh(core_axis_name="c", subcore_axis_name="s")

    @functools.partial(
        pl.kernel,
        mesh=mesh,
        out_type=jax.ShapeDtypeStruct((B, _EDIM, hw), jnp.float32),
        compiler_params=pltpu.CompilerParams(needs_layout_passes=False),
        scratch_types=[
            pltpu.VMEM((_NUM_EMB * _EDIM,), jnp.float32),
            pltpu.VMEM((bpw,), jnp.int32),
            pltpu.VMEM((_EDIM, bpw), jnp.float32),
            pltpu.SemaphoreType.DMA,
            pltpu.SemaphoreType.DMA,
        ],
    )
    def gather_k(table_hbm, idx_hbm, out_hbm, tab_v, idx_v, out_v, sem_t, sem_i):
        wid = lax.axis_index("s") * _NC + lax.axis_index("c")
        base = wid * bpw
        b = base // hw
        off = base % hw
        cp_t = pltpu.async_copy(table_hbm, tab_v, sem_t)
        cp_i = pltpu.async_copy(idx_hbm.at[pl.ds(base, bpw)], idx_v, sem_i)
        cp_t.wait()
        cp_i.wait()

        iota16 = lax.iota(jnp.int32, 16)

        @plsc.parallel_loop(0, bpw // 16, unroll=4)
        def _(jc):
            idx16 = idx_v[pl.ds(jc * 16, 16)]
            rowa = idx16 * _EDIM
            jvec = jc * 16 + iota16
            for c in range(_EDIM):
                cvec = (iota16 + c) & (_EDIM - 1)
                vals = plsc.load_gather(tab_v, [rowa + cvec])
                plsc.store_scatter(out_v, [cvec, jvec], vals)
        pltpu.sync_copy(out_v, out_hbm.at[b, :, pl.ds(off, bpw)])

    return gather_k


def kernel(inputs, embeddings):
    B, C, H, W = inputs.shape
    hw = H * W
    x = inputs.reshape(B, C, hw)

    idx3, loss_sum = _argmin_call(x, embeddings)
    idx_flat = idx3.reshape(B * hw)

    q_cm = _make_sc_gather(B, hw)(
        embeddings.reshape(_NUM_EMB * _EDIM), idx_flat)

    quantized = q_cm.reshape(B, C, H, W)
    loss = (1.0 + _BETA) * loss_sum[0, 0] / inputs.size
    return (quantized, loss)
